# baseline (speedup 1.0000x reference)
import functools

import jax
import jax.numpy as jnp
from jax import lax
from jax.experimental import pallas as pl
from jax.experimental.pallas import tpu as pltpu
from jax.experimental.pallas import tpu_sc as plsc

_L = 16


@functools.lru_cache(maxsize=None)
def _make_sc_permute(B, C, D):
    info = plsc.get_sparse_core_info()
    NC, NS = info.num_cores, info.num_subcores
    NW = NC * NS
    assert C % _L == 0 and B % NW == 0
    batches_per_tile = B // NW
    n_chunks = C // _L
    mesh = plsc.VectorSubcoreMesh(core_axis_name="c", subcore_axis_name="s")

    @functools.partial(
        pl.kernel,
        mesh=mesh,
        compiler_params=pltpu.CompilerParams(
            needs_layout_passes=False, use_tc_tiling_on_sc=False
        ),
        out_type=jax.ShapeDtypeStruct((B * C, D), jnp.float32),
        scratch_types=[
            pltpu.VMEM((C,), jnp.float32),
            pltpu.VMEM((C,), jnp.int32),
            pltpu.VMEM((_L, D), jnp.float32),
            pltpu.SemaphoreType.DMA,
        ],
    )
    def k(indexes_hbm, in_hbm, out_hbm, idxs_v, sel_v, buf, gsem):
        wid = lax.axis_index("s") * NC + lax.axis_index("c")
        pltpu.sync_copy(indexes_hbm, idxs_v)

        iota = lax.iota(jnp.int32, _L)
        one = jnp.int32(1)
        zero = jnp.int32(0)

        acc = jnp.zeros((_L,), jnp.int32)
        for c in range(n_chunks):
            v = idxs_v[pl.ds(c * _L, _L)]
            acc = acc + jnp.where(v != 0.0, one, zero)
        total_nz = zero
        for j in range(_L):
            total_nz = total_nz + acc[j]

        nz = zero
        z = zero
        for c in range(n_chunks):
            v = idxs_v[pl.ds(c * _L, _L)]
            posvec = jnp.zeros((_L,), jnp.int32)
            for j in range(_L):
                mj = v[j] != 0.0
                pos_j = jnp.where(mj, nz, total_nz + z)
                posvec = jnp.where(iota == j, pos_j, posvec)
                nz = nz + jnp.where(mj, one, zero)
                z = z + jnp.where(mj, zero, one)
            plsc.store_scatter(sel_v, [posvec], iota + (c * _L))

        for b in range(batches_per_tile):
            row0 = (wid * batches_per_tile + b) * C
            for c in range(n_chunks):
                src_rows = sel_v[pl.ds(c * _L, _L)] + row0
                pltpu.async_copy(in_hbm.at[src_rows], buf, gsem).wait()
                pltpu.sync_copy(buf, out_hbm.at[pl.ds(row0 + c * _L, _L)])

    return k


def kernel(input_tensor, indexes):
    B, C, H, W = input_tensor.shape
    flat = input_tensor.reshape(B * C, H * W)
    out = _make_sc_permute(B, C, H * W)(indexes, flat)
    return out.reshape(B, C, H, W)


# speedup vs baseline: 1.0120x; 1.0120x over previous
import functools

import jax
import jax.numpy as jnp
from jax import lax
from jax.experimental import pallas as pl
from jax.experimental.pallas import tpu as pltpu
from jax.experimental.pallas import tpu_sc as plsc

_L = 16


@functools.lru_cache(maxsize=None)
def _make_sc_permute(B, C, D):
    info = plsc.get_sparse_core_info()
    NC, NS = info.num_cores, info.num_subcores
    NW = NC * NS
    assert C % _L == 0 and B % NW == 0
    batches_per_tile = B // NW
    n_chunks = C // _L
    mesh = plsc.VectorSubcoreMesh(core_axis_name="c", subcore_axis_name="s")

    @functools.partial(
        pl.kernel,
        mesh=mesh,
        compiler_params=pltpu.CompilerParams(
            needs_layout_passes=False, use_tc_tiling_on_sc=False
        ),
        out_type=jax.ShapeDtypeStruct((B * C, D), jnp.float32),
        scratch_types=[
            pltpu.VMEM((C,), jnp.float32),
            pltpu.VMEM((C,), jnp.int32),
            pltpu.VMEM((_L, D), jnp.float32),
            pltpu.VMEM((_L, D), jnp.float32),
            pltpu.SemaphoreType.DMA,
            pltpu.SemaphoreType.DMA,
            pltpu.SemaphoreType.DMA,
            pltpu.SemaphoreType.DMA,
        ],
    )
    def k(indexes_hbm, in_hbm, out_hbm, idxs_v, sel_v, buf0, buf1, g0, g1, p0, p1):
        wid = lax.axis_index("s") * NC + lax.axis_index("c")
        pltpu.sync_copy(indexes_hbm, idxs_v)

        iota = lax.iota(jnp.int32, _L)
        one = jnp.int32(1)
        zero = jnp.int32(0)

        acc = jnp.zeros((_L,), jnp.int32)
        for c in range(n_chunks):
            v = idxs_v[pl.ds(c * _L, _L)]
            acc = acc + jnp.where(v != 0.0, one, zero)
        total_nz = zero
        for j in range(_L):
            total_nz = total_nz + acc[j]

        nz = zero
        z = zero
        for c in range(n_chunks):
            v = idxs_v[pl.ds(c * _L, _L)]
            posvec = jnp.zeros((_L,), jnp.int32)
            for j in range(_L):
                mj = v[j] != 0.0
                pos_j = jnp.where(mj, nz, total_nz + z)
                posvec = jnp.where(iota == j, pos_j, posvec)
                nz = nz + jnp.where(mj, one, zero)
                z = z + jnp.where(mj, zero, one)
            plsc.store_scatter(sel_v, [posvec], iota + (c * _L))

        bufs = (buf0, buf1)
        gsems = (g0, g1)
        psems = (p0, p1)
        nsteps = batches_per_tile * n_chunks
        gathers = [None] * nsteps
        writes = [None] * nsteps
        for i in range(nsteps):
            b, c = divmod(i, n_chunks)
            row0 = (wid * batches_per_tile + b) * C
            if i >= 2:
                writes[i - 2].wait()
            src_rows = sel_v[pl.ds(c * _L, _L)] + row0
            gathers[i] = pltpu.async_copy(
                in_hbm.at[src_rows], bufs[i % 2], gsems[i % 2]
            )
            if i >= 1:
                pb, pc = divmod(i - 1, n_chunks)
                prow0 = (wid * batches_per_tile + pb) * C
                gathers[i - 1].wait()
                writes[i - 1] = pltpu.async_copy(
                    bufs[(i - 1) % 2],
                    out_hbm.at[pl.ds(prow0 + pc * _L, _L)],
                    psems[(i - 1) % 2],
                )
        lb, lc = divmod(nsteps - 1, n_chunks)
        lrow0 = (wid * batches_per_tile + lb) * C
        gathers[nsteps - 1].wait()
        writes[nsteps - 1] = pltpu.async_copy(
            bufs[(nsteps - 1) % 2],
            out_hbm.at[pl.ds(lrow0 + lc * _L, _L)],
            psems[(nsteps - 1) % 2],
        )
        writes[nsteps - 2].wait()
        writes[nsteps - 1].wait()

    return k


def kernel(input_tensor, indexes):
    B, C, H, W = input_tensor.shape
    flat = input_tensor.reshape(B * C, H * W)
    out = _make_sc_permute(B, C, H * W)(indexes, flat)
    return out.reshape(B, C, H, W)
